# baseline (speedup 1.0000x reference)
import functools

import jax
import jax.numpy as jnp
from jax import lax
from jax.experimental import pallas as pl
from jax.experimental.pallas import tpu as pltpu
from jax.experimental.pallas import tpu_sc as plsc

B, N, C = 16, 4096, 512
K = N // 2
T = 512
IT = N // T
KT = K // T


def _rank_body(si_ref, sj_ref, rank_ref):
    it = pl.program_id(1)
    jt = pl.program_id(2)

    @pl.when(jt == 0)
    def _():
        rank_ref[...] = jnp.zeros_like(rank_ref)

    u_i = lax.bitcast_convert_type(
        si_ref[...].reshape(1, T), jnp.int32) * 2
    u_j = lax.bitcast_convert_type(
        sj_ref[...].reshape(T, 1), jnp.int32) * 2

    def count(thresh):
        cnt = (thresh > u_i).astype(jnp.int32)
        part = jnp.sum(cnt, axis=0, keepdims=True)
        rank_ref[...] += part.reshape(1, 1, T)

    @pl.when(it == jt)
    def _():
        ig = lax.broadcasted_iota(jnp.int32, (1, T), 1)
        jg = lax.broadcasted_iota(jnp.int32, (T, 1), 0)
        count(u_j + jnp.where(jg < ig, 1, 0))

    @pl.when(it != jt)
    def _():
        count(u_j + jnp.where(jt < it, 1, 0))


def _gather_body(rank_ref, s_ref, h_ref, out_ref):
    jt = pl.program_id(1)

    @pl.when(jt == 0)
    def _():
        out_ref[...] = jnp.zeros_like(out_ref)

    rank_row = rank_ref[...].reshape(1, T)
    s_row = s_ref[...].reshape(1, T)
    hmat = h_ref[0]
    for kt in range(KT):
        kio = kt * T + lax.broadcasted_iota(jnp.int32, (T, 1), 0)
        pmat = jnp.where(rank_row == kio, s_row, 0.0)
        out_ref[0, kt * T:(kt + 1) * T, :] += lax.dot_general(
            pmat, hmat, (((1,), (0,)), ((), ())),
            preferred_element_type=jnp.float32)


NW = 32
NPW = B * N // NW
CH = 128
NCH = NPW // CH
PAD = 512
BKP = B * K + PAD


def _sc_scatter_body(rank_hbm, s_hbm, h_hbm, out_hbm, ssel_hbm,
                     rank_v, s_v, tgt2, rows2, sem):
    cc_ = jax.lax.axis_index("c")
    ss_ = jax.lax.axis_index("s")
    wid = ss_ * 2 + cc_
    b = wid // 2
    half = wid % 2
    node_base = b * N + half * NPW

    pltpu.sync_copy(rank_hbm.at[pl.ds(node_base, NPW)], rank_v)
    pltpu.sync_copy(s_hbm.at[pl.ds(node_base, NPW)], s_v)

    lane = jax.lax.broadcasted_iota(jnp.int32, (16,), 0)

    for r in range(NCH):
        row = tgt2.at[r]
        for j in range(CH // 16):
            rvec = rank_v[pl.ds(r * CH + j * 16, 16)]
            sel = rvec < K
            row[pl.ds(j * 16, 16)] = jnp.where(sel, b * K + rvec,
                                               B * K + wid)

    for ch in range(NCH):
        pltpu.async_copy(
            h_hbm.at[pl.ds(node_base + ch * CH, CH)], rows2, sem).wait()
        pltpu.sync_copy(rows2, out_hbm.at[tgt2.at[ch]])
        pltpu.sync_copy(s_v.at[pl.ds(ch * CH, CH)], ssel_hbm.at[tgt2.at[ch]])


_sc_scatter = functools.partial(
    pl.kernel,
    mesh=plsc.VectorSubcoreMesh(core_axis_name="c", subcore_axis_name="s"),
    out_type=[
        jax.ShapeDtypeStruct((BKP, C), jnp.float32),
        jax.ShapeDtypeStruct((BKP,), jnp.float32),
    ],
    scratch_types=[
        pltpu.VMEM((NPW,), jnp.int32),
        pltpu.VMEM((NPW,), jnp.float32),
        pltpu.VMEM((NCH, CH), jnp.int32),
        pltpu.VMEM((CH, C), jnp.float32),
        pltpu.SemaphoreType.DMA,
    ],
)(_sc_scatter_body)


def _scale_body(raw_ref, s_ref, out_ref):
    out_ref[...] = raw_ref[...] * s_ref[...]


@jax.jit
def kernel(h, W, b):
    scores = jax.nn.sigmoid(jnp.einsum('bnc,oc->bno', h, W) + b)
    s_row = scores.reshape(B, 1, N)

    ranks = pl.pallas_call(
        _rank_body,
        grid=(B, IT, IT),
        in_specs=[
            pl.BlockSpec((1, 1, T), lambda b_, i, j: (b_, 0, i)),
            pl.BlockSpec((1, 1, T), lambda b_, i, j: (b_, 0, j)),
        ],
        out_specs=pl.BlockSpec((1, 1, T), lambda b_, i, j: (b_, 0, i)),
        out_shape=jax.ShapeDtypeStruct((B, 1, N), jnp.int32),
    )(s_row, s_row)

    rank_flat = ranks.reshape(B * N)
    s_flat = scores.reshape(B * N)
    h_flat = h.reshape(B * N, C)
    out_raw, s_sel = _sc_scatter(rank_flat, s_flat, h_flat)
    s_sel2 = s_sel.reshape(BKP, 1)

    SB = 512
    out = pl.pallas_call(
        _scale_body,
        grid=(B * K // SB,),
        in_specs=[
            pl.BlockSpec((SB, C), lambda i: (i, 0)),
            pl.BlockSpec((SB, 1), lambda i: (i, 0)),
        ],
        out_specs=pl.BlockSpec((SB, C), lambda i: (i, 0)),
        out_shape=jax.ShapeDtypeStruct((B * K, C), jnp.float32),
    )(out_raw, s_sel2)
    return out.reshape(B, K, C)


# speedup vs baseline: 2.9142x; 2.9142x over previous
import functools

import jax
import jax.numpy as jnp
from jax import lax
from jax.experimental import pallas as pl
from jax.experimental.pallas import tpu as pltpu
from jax.experimental.pallas import tpu_sc as plsc

B, N, C = 16, 4096, 512
K = N // 2
T = 512
IT = N // T
KT = K // T


def _rank_body(si_ref, sj_ref, rank_ref):
    it = pl.program_id(1)
    jt = pl.program_id(2)

    @pl.when(jt == 0)
    def _():
        rank_ref[...] = jnp.zeros_like(rank_ref)

    u_i = lax.bitcast_convert_type(
        si_ref[...].reshape(1, T), jnp.int32) * 2
    u_j = lax.bitcast_convert_type(
        sj_ref[...].reshape(T, 1), jnp.int32) * 2

    def count(thresh):
        cnt = (thresh > u_i).astype(jnp.int32)
        part = jnp.sum(cnt, axis=0, keepdims=True)
        rank_ref[...] += part.reshape(1, 1, T)

    @pl.when(it == jt)
    def _():
        ig = lax.broadcasted_iota(jnp.int32, (1, T), 1)
        jg = lax.broadcasted_iota(jnp.int32, (T, 1), 0)
        count(u_j + jnp.where(jg < ig, 1, 0))

    @pl.when(it != jt)
    def _():
        count(u_j + jnp.where(jt < it, 1, 0))


def _gather_body(rank_ref, s_ref, h_ref, out_ref):
    jt = pl.program_id(1)

    @pl.when(jt == 0)
    def _():
        out_ref[...] = jnp.zeros_like(out_ref)

    rank_row = rank_ref[...].reshape(1, T)
    s_row = s_ref[...].reshape(1, T)
    hmat = h_ref[0]
    for kt in range(KT):
        kio = kt * T + lax.broadcasted_iota(jnp.int32, (T, 1), 0)
        pmat = jnp.where(rank_row == kio, s_row, 0.0)
        out_ref[0, kt * T:(kt + 1) * T, :] += lax.dot_general(
            pmat, hmat, (((1,), (0,)), ((), ())),
            preferred_element_type=jnp.float32)


NW = 32
NPW = B * N // NW
CH = 128
NCH = NPW // CH
PAD = 512
BKP = B * K + PAD


def _sc_scatter_body(rank_hbm, s_hbm, h_hbm, out_hbm, ssel_hbm,
                     rank_v, s_v, tgt2, rows2, sem):
    cc_ = jax.lax.axis_index("c")
    ss_ = jax.lax.axis_index("s")
    wid = ss_ * 2 + cc_
    b = wid // 2
    half = wid % 2
    node_base = b * N + half * NPW

    pltpu.sync_copy(rank_hbm.at[pl.ds(node_base, NPW)], rank_v)
    pltpu.sync_copy(s_hbm.at[pl.ds(node_base, NPW)], s_v)

    lane = jax.lax.broadcasted_iota(jnp.int32, (16,), 0)

    for r in range(NCH):
        row = tgt2.at[r]
        for j in range(CH // 16):
            rvec = rank_v[pl.ds(r * CH + j * 16, 16)]
            sel = rvec < K
            trash = B * K + ((r * CH + j * 16 + lane * 37 + wid * 16) & (PAD - 1))
            row[pl.ds(j * 16, 16)] = jnp.where(sel, b * K + rvec, trash)

    for ch in range(NCH):
        pltpu.async_copy(
            h_hbm.at[pl.ds(node_base + ch * CH, CH)], rows2, sem).wait()
        pltpu.sync_copy(rows2, out_hbm.at[tgt2.at[ch]])
        pltpu.sync_copy(s_v.at[pl.ds(ch * CH, CH)], ssel_hbm.at[tgt2.at[ch]])


_sc_scatter = functools.partial(
    pl.kernel,
    mesh=plsc.VectorSubcoreMesh(core_axis_name="c", subcore_axis_name="s"),
    out_type=[
        jax.ShapeDtypeStruct((BKP, C), jnp.float32),
        jax.ShapeDtypeStruct((BKP,), jnp.float32),
    ],
    scratch_types=[
        pltpu.VMEM((NPW,), jnp.int32),
        pltpu.VMEM((NPW,), jnp.float32),
        pltpu.VMEM((NCH, CH), jnp.int32),
        pltpu.VMEM((CH, C), jnp.float32),
        pltpu.SemaphoreType.DMA,
    ],
)(_sc_scatter_body)


def _scale_body(raw_ref, s_ref, out_ref):
    out_ref[...] = raw_ref[...] * s_ref[...]


@jax.jit
def kernel(h, W, b):
    scores = jax.nn.sigmoid(jnp.einsum('bnc,oc->bno', h, W) + b)
    s_row = scores.reshape(B, 1, N)

    ranks = pl.pallas_call(
        _rank_body,
        grid=(B, IT, IT),
        in_specs=[
            pl.BlockSpec((1, 1, T), lambda b_, i, j: (b_, 0, i)),
            pl.BlockSpec((1, 1, T), lambda b_, i, j: (b_, 0, j)),
        ],
        out_specs=pl.BlockSpec((1, 1, T), lambda b_, i, j: (b_, 0, i)),
        out_shape=jax.ShapeDtypeStruct((B, 1, N), jnp.int32),
    )(s_row, s_row)

    rank_flat = ranks.reshape(B * N)
    s_flat = scores.reshape(B * N)
    h_flat = h.reshape(B * N, C)
    out_raw, s_sel = _sc_scatter(rank_flat, s_flat, h_flat)
    s_sel2 = s_sel.reshape(BKP, 1)

    SB = 512
    out = pl.pallas_call(
        _scale_body,
        grid=(B * K // SB,),
        in_specs=[
            pl.BlockSpec((SB, C), lambda i: (i, 0)),
            pl.BlockSpec((SB, 1), lambda i: (i, 0)),
        ],
        out_specs=pl.BlockSpec((SB, C), lambda i: (i, 0)),
        out_shape=jax.ShapeDtypeStruct((B * K, C), jnp.float32),
    )(out_raw, s_sel2)
    return out.reshape(B, K, C)


# speedup vs baseline: 7.7520x; 2.6601x over previous
import functools

import jax
import jax.numpy as jnp
from jax import lax
from jax.experimental import pallas as pl

B, N, C = 16, 4096, 512
K = N // 2
T = 512
IT = N // T
KT = K // T


def _rank_body(si_ref, sj_ref, rank_ref):
    it = pl.program_id(1)
    jt = pl.program_id(2)

    @pl.when(jt == 0)
    def _():
        rank_ref[...] = jnp.zeros_like(rank_ref)

    u_i = lax.bitcast_convert_type(
        si_ref[...].reshape(1, T), jnp.int32) * 2
    u_j = lax.bitcast_convert_type(
        sj_ref[...].reshape(T, 1), jnp.int32) * 2

    def count(thresh):
        cnt = (thresh > u_i).astype(jnp.int32)
        part = jnp.sum(cnt, axis=0, keepdims=True)
        rank_ref[...] += part.reshape(1, 1, T)

    @pl.when(it == jt)
    def _():
        ig = lax.broadcasted_iota(jnp.int32, (1, T), 1)
        jg = lax.broadcasted_iota(jnp.int32, (T, 1), 0)
        count(u_j + jnp.where(jg < ig, 1, 0))

    @pl.when(it != jt)
    def _():
        count(u_j + jnp.where(jt < it, 1, 0))


def _gather_body(rank_ref, s_ref, h_ref, out_ref):
    jt = pl.program_id(1)

    @pl.when(jt == 0)
    def _():
        out_ref[...] = jnp.zeros_like(out_ref)

    rank_row = rank_ref[...].reshape(1, T)
    s_row = s_ref[...].reshape(1, T)
    hmat = h_ref[0]
    for kt in range(KT):
        kio = kt * T + lax.broadcasted_iota(jnp.int32, (T, 1), 0)
        pmat = jnp.where(rank_row == kio, s_row, 0.0)
        out_ref[0, kt * T:(kt + 1) * T, :] += lax.dot_general(
            pmat, hmat, (((1,), (0,)), ((), ())),
            preferred_element_type=jnp.float32)


@jax.jit
def kernel(h, W, b):
    scores = jax.nn.sigmoid(jnp.einsum('bnc,oc->bno', h, W) + b)
    s_row = scores.reshape(B, 1, N)

    ranks = pl.pallas_call(
        _rank_body,
        grid=(B, IT, IT),
        in_specs=[
            pl.BlockSpec((1, 1, T), lambda b_, i, j: (b_, 0, i)),
            pl.BlockSpec((1, 1, T), lambda b_, i, j: (b_, 0, j)),
        ],
        out_specs=pl.BlockSpec((1, 1, T), lambda b_, i, j: (b_, 0, i)),
        out_shape=jax.ShapeDtypeStruct((B, 1, N), jnp.int32),
    )(s_row, s_row)

    out = pl.pallas_call(
        _gather_body,
        grid=(B, IT),
        in_specs=[
            pl.BlockSpec((1, 1, T), lambda b_, j: (b_, 0, j)),
            pl.BlockSpec((1, 1, T), lambda b_, j: (b_, 0, j)),
            pl.BlockSpec((1, T, C), lambda b_, j: (b_, j, 0)),
        ],
        out_specs=pl.BlockSpec((1, K, C), lambda b_, j: (b_, 0, 0)),
        out_shape=jax.ShapeDtypeStruct((B, K, C), jnp.float32),
    )(ranks, s_row, h)
    return out
